# baseline (speedup 1.0000x reference)
import functools

import jax
import jax.numpy as jnp
from jax import lax
from jax.experimental import pallas as pl
from jax.experimental.pallas import tpu as pltpu
from jax.experimental.pallas import tpu_sc as plsc

N = 100000
D = 128
V = 128
OUT = 128
G = 64
H2 = 256

TILE = 20000
T = N // TILE

NW = 32
CH = 800
NCH = N // CH


def _sc_counts_body(batch_hbm, maskf_hbm, out_hbm, bbuf, mbuf, cnt_v):
    wid = lax.axis_index("s") * 2 + lax.axis_index("c")
    lane = jax.lax.broadcasted_iota(jnp.int32, (16,), 0)
    for r in range(128):
        cnt_v[r] = jnp.zeros((16,), jnp.float32)
    for k in range((NCH + NW - 1) // NW):
        c = wid + NW * k

        @pl.when(c < NCH)
        def _chunk():
            base = c * CH
            pltpu.sync_copy(batch_hbm.at[pl.ds(base, CH)], bbuf)
            pltpu.sync_copy(maskf_hbm.at[pl.ds(base, CH)], mbuf)
            for j in range(CH // 16):
                ids = bbuf[pl.ds(j * 16, 16)]
                vals = mbuf[pl.ds(j * 16, 16)]
                plsc.addupdate_scatter(cnt_v, [ids, lane], vals)

    pltpu.sync_copy(cnt_v, out_hbm.at[wid])


_sc_counts = functools.partial(
    pl.kernel,
    mesh=plsc.VectorSubcoreMesh(core_axis_name="c", subcore_axis_name="s"),
    out_type=jax.ShapeDtypeStruct((NW, 128, 16), jnp.float32),
    scratch_types=[
        pltpu.VMEM((CH,), jnp.int32),
        pltpu.VMEM((CH,), jnp.float32),
        pltpu.VMEM((128, 16), jnp.float32),
    ],
    compiler_params=pltpu.CompilerParams(needs_layout_passes=False),
)(_sc_counts_body)


def _main_body(x_ref, m_ref, b_ref, Wm_ref, bm_ref, acc_ref):
    i = pl.program_id(0)

    @pl.when(i == 0)
    def _init():
        acc_ref[...] = jnp.zeros_like(acc_ref)

    x = x_ref[...]
    h = jnp.dot(x, Wm_ref[...], preferred_element_type=jnp.float32)
    h = jnp.maximum(h + bm_ref[...], 0.0)

    m = m_ref[0, 0, :]
    b = b_ref[0, 0, :]
    seg = jax.lax.broadcasted_iota(jnp.int32, (G, TILE), 0)
    S = jnp.where(seg == b[None, :], m[None, :], 0.0)
    acc_ref[...] += jnp.dot(S, h, preferred_element_type=jnp.float32)


def _head_body(acc_ref, part_ref, W1_ref, b1_ref, W2_ref, b2_ref, out_ref):
    cnt = jnp.sum(part_ref[...], axis=(0, 2))[:G]
    mean = acc_ref[...] / jnp.maximum(cnt[:, None], 1.0)
    hid = jnp.dot(mean, W1_ref[...], preferred_element_type=jnp.float32)
    hid = jnp.maximum(hid + b1_ref[...], 0.0)
    out = jnp.dot(hid, W2_ref[...], preferred_element_type=jnp.float32)
    out_ref[...] = out + b2_ref[...]


@jax.jit
def kernel(x, mask, batch, Wm, bm, W1, b1, W2, b2):
    maskf = mask.astype(jnp.float32)
    maskf3 = maskf.reshape(T, 1, TILE)
    batch3 = batch.reshape(T, 1, TILE)
    bm2 = bm.reshape(1, V)
    b12 = b1.reshape(1, H2)
    b22 = b2.reshape(1, OUT)

    partials = _sc_counts(batch, maskf)

    full = lambda shape: pl.BlockSpec(shape, lambda i: (0,) * len(shape))
    acc = pl.pallas_call(
        _main_body,
        grid=(T,),
        in_specs=[
            pl.BlockSpec((TILE, D), lambda i: (i, 0)),
            pl.BlockSpec((1, 1, TILE), lambda i: (i, 0, 0)),
            pl.BlockSpec((1, 1, TILE), lambda i: (i, 0, 0)),
            full((D, V)),
            full((1, V)),
        ],
        out_specs=full((G, V)),
        out_shape=jax.ShapeDtypeStruct((G, V), jnp.float32),
        compiler_params=pltpu.CompilerParams(
            dimension_semantics=("arbitrary",),
        ),
    )(x, maskf3, batch3, Wm, bm2)

    nofull = lambda shape: pl.BlockSpec(shape, lambda: (0,) * len(shape))
    out = pl.pallas_call(
        _head_body,
        in_specs=[nofull((G, V)), nofull((NW, 128, 16)), nofull((V, H2)),
                  nofull((1, H2)), nofull((H2, OUT)), nofull((1, OUT))],
        out_specs=nofull((G, OUT)),
        out_shape=jax.ShapeDtypeStruct((G, OUT), jnp.float32),
    )(acc, partials, W1, b12, W2, b22)
    return out


# speedup vs baseline: 1.7193x; 1.7193x over previous
import jax
import jax.numpy as jnp
from jax.experimental import pallas as pl
from jax.experimental.pallas import tpu as pltpu

N = 100000
D = 128
V = 128
OUT = 128
G = 64
H2 = 256

TILE = 20000
T = N // TILE


def _body(x_ref, m_ref, b_ref, Wm_ref, bm_ref, W1_ref, b1_ref, W2_ref,
          b2_ref, out_ref, acc_ref, cnt_ref):
    i = pl.program_id(0)

    @pl.when(i == 0)
    def _init():
        acc_ref[...] = jnp.zeros_like(acc_ref)
        cnt_ref[...] = jnp.zeros_like(cnt_ref)

    x = x_ref[...]
    h = jnp.dot(x, Wm_ref[...], preferred_element_type=jnp.float32)
    h = jnp.maximum(h + bm_ref[...], 0.0)

    m = m_ref[0, 0, :]
    b = b_ref[0, 0, :]
    seg = jax.lax.broadcasted_iota(jnp.int32, (G, TILE), 0)
    S = jnp.where(seg == b[None, :], m[None, :], 0.0)
    acc_ref[...] += jnp.dot(S, h, preferred_element_type=jnp.float32)
    cnt_ref[...] += jnp.sum(S, axis=1, keepdims=True)

    @pl.when(i == T - 1)
    def _head():
        mean = acc_ref[...] / jnp.maximum(cnt_ref[...], 1.0)
        hid = jnp.dot(mean, W1_ref[...], preferred_element_type=jnp.float32)
        hid = jnp.maximum(hid + b1_ref[...], 0.0)
        out = jnp.dot(hid, W2_ref[...], preferred_element_type=jnp.float32)
        out_ref[...] = out + b2_ref[...]


@jax.jit
def kernel(x, mask, batch, Wm, bm, W1, b1, W2, b2):
    maskf = mask.astype(jnp.float32).reshape(T, 1, TILE)
    batch3 = batch.reshape(T, 1, TILE)
    bm2 = bm.reshape(1, V)
    b12 = b1.reshape(1, H2)
    b22 = b2.reshape(1, OUT)

    full = lambda shape: pl.BlockSpec(shape, lambda i: (0,) * len(shape))
    out = pl.pallas_call(
        _body,
        grid=(T,),
        in_specs=[
            pl.BlockSpec((TILE, D), lambda i: (i, 0)),
            pl.BlockSpec((1, 1, TILE), lambda i: (i, 0, 0)),
            pl.BlockSpec((1, 1, TILE), lambda i: (i, 0, 0)),
            full((D, V)),
            full((1, V)),
            full((V, H2)),
            full((1, H2)),
            full((H2, OUT)),
            full((1, OUT)),
        ],
        out_specs=full((G, OUT)),
        out_shape=jax.ShapeDtypeStruct((G, OUT), jnp.float32),
        scratch_shapes=[
            pltpu.VMEM((G, V), jnp.float32),
            pltpu.VMEM((G, V), jnp.float32),
        ],
        compiler_params=pltpu.CompilerParams(
            dimension_semantics=("arbitrary",),
        ),
    )(x, maskf, batch3, Wm, bm2, W1, b12, W2, b22)
    return out
